# Initial kernel scaffold; baseline (speedup 1.0000x reference)
import functools

import jax
import jax.numpy as jnp
from jax import lax
from jax.experimental import pallas as pl
from jax.experimental.pallas import tpu as pltpu
from jax.experimental.pallas import tpu_sc as plsc

N = 10000
NPAD = 10240
E = 320000
F = 128
G = 64
NC, NS = 2, 16
NW = NC * NS
EPB = 128
CH = 80
EPAD = NW * EPB * CH
RPT = NPAD // NS
BLK = 512
BI = 320
WIN = 3 * BI
NB = NPAD // BI


_agg_out = (
    jax.ShapeDtypeStruct((NC, NPAD, F), jnp.float32),
    jax.ShapeDtypeStruct((NC, NPAD, 16), jnp.float32),
)


@functools.partial(
    pl.kernel,
    out_type=_agg_out,
    mesh=plsc.VectorSubcoreMesh(core_axis_name="c", subcore_axis_name="s"),
    scratch_types=[
        pltpu.VMEM((EPB,), jnp.int32),
        pltpu.VMEM((EPB,), jnp.int32),
        pltpu.VMEM((EPB, F), jnp.float32),
        pltpu.VMEM((EPB, 16), jnp.float32),
        pltpu.VMEM_SHARED((NPAD, F), jnp.float32),
        pltpu.VMEM_SHARED((NPAD, 16), jnp.float32),
        pltpu.SemaphoreType.DMA,
        pltpu.SemaphoreType.DMA,
    ],
)
def _sc_aggregate(h, srcp, dstp, kept16, zrow, zdeg, outs, outd,
                  idx_v, dst_v, rows_v, kf_v, acc_sh, deg_sh, gsem, ksem):
  c = lax.axis_index("c")
  s = lax.axis_index("s")
  wid = c * NS + s
  r0 = s * RPT
  pltpu.sync_copy(zrow.at[pl.ds(r0, RPT)], acc_sh.at[pl.ds(r0, RPT)])
  pltpu.sync_copy(zdeg.at[pl.ds(r0, RPT)], deg_sh.at[pl.ds(r0, RPT)])
  plsc.subcore_barrier()
  ebase = wid * (CH * EPB)

  def chunk(i, carry):
    base = ebase + i * EPB
    pltpu.sync_copy(srcp.at[pl.ds(base, EPB)], idx_v)
    pltpu.sync_copy(dstp.at[pl.ds(base, EPB)], dst_v)
    pltpu.async_copy(h.at[idx_v], rows_v, gsem).wait()
    pltpu.async_copy(kept16.at[idx_v], kf_v, ksem).wait()
    pltpu.sync_copy(rows_v, acc_sh.at[dst_v], add=True)
    pltpu.sync_copy(kf_v, deg_sh.at[dst_v], add=True)
    return carry

  lax.fori_loop(0, CH, chunk, 0)
  plsc.subcore_barrier()
  pltpu.sync_copy(acc_sh.at[pl.ds(r0, RPT)], outs.at[c, pl.ds(r0, RPT)])
  pltpu.sync_copy(deg_sh.at[pl.ds(r0, RPT)], outd.at[c, pl.ds(r0, RPT)])


def _layer_body(s2, d2, kept, h, wl, bl, wr, pw, hmid_ref, score_ref):
  ssum = s2[0] + s2[1]
  deg = d2[0, :, 0:1] + d2[1, :, 0:1]
  keptc = kept[:, 0:1]
  mean = ssum * keptc / jnp.maximum(deg, 1.0)
  z = (lax.dot_general(mean, wl[...], (((1,), (1,)), ((), ())),
                       preferred_element_type=jnp.float32)
       + bl[...]
       + lax.dot_general(h[...], wr[...], (((1,), (1,)), ((), ())),
                         preferred_element_type=jnp.float32))
  hmid = jnp.maximum(z, 0.0)
  hmid_ref[...] = hmid
  pwv = pw[...]
  nrm = jnp.sqrt(jnp.sum(pwv * pwv))
  score_ref[...] = jnp.tanh(
      jnp.sum(hmid * pwv, axis=1, keepdims=True) / nrm)


def _layer_step(s2, d2, kept16, h, wl, bl, wr, pw):
  return pl.pallas_call(
      _layer_body,
      grid=(NPAD // BLK,),
      in_specs=[
          pl.BlockSpec((NC, BLK, F), lambda i: (0, i, 0)),
          pl.BlockSpec((NC, BLK, 16), lambda i: (0, i, 0)),
          pl.BlockSpec((BLK, 16), lambda i: (i, 0)),
          pl.BlockSpec((BLK, F), lambda i: (i, 0)),
          pl.BlockSpec((F, F), lambda i: (0, 0)),
          pl.BlockSpec((1, F), lambda i: (0, 0)),
          pl.BlockSpec((F, F), lambda i: (0, 0)),
          pl.BlockSpec((1, F), lambda i: (0, 0)),
      ],
      out_specs=[
          pl.BlockSpec((BLK, F), lambda i: (i, 0)),
          pl.BlockSpec((BLK, 1), lambda i: (i, 0)),
      ],
      out_shape=[
          jax.ShapeDtypeStruct((NPAD, F), jnp.float32),
          jax.ShapeDtypeStruct((NPAD, 1), jnp.float32),
      ],
  )(s2, d2, kept16, h, wl, bl, wr, pw)


def _rank_body(srow, prow, brow, arow, scol, pcol, bcol, acol,
               rank_ref, counts_ref):
  i = pl.program_id(0)
  start = jnp.clip(i * BI - BI, 0, NPAD - WIN)
  sj = srow[:, pl.ds(start, WIN)]
  pj = prow[:, pl.ds(start, WIN)]
  bj = brow[:, pl.ds(start, WIN)]
  aj = arow[:, pl.ds(start, WIN)]
  si = scol[...]
  pi = pcol[...]
  bi = bcol[...]
  beats = (sj > si) | ((sj == si) & (pj < pi))
  valid = (bj == bi) & (aj > 0.5)
  rank_ref[...] = jnp.sum(
      jnp.where(beats & valid, 1.0, 0.0), axis=1, keepdims=True)
  gids = lax.broadcasted_iota(jnp.float32, (1, G), 1)
  cnt = jnp.sum(jnp.where((bi == gids) & (acol[...] > 0.5), 1.0, 0.0),
                axis=0, keepdims=True)

  @pl.when(i == 0)
  def _():
    counts_ref[...] = jnp.zeros_like(counts_ref)

  counts_ref[...] += cnt


def _rank_step(srow, prow, brow, arow, scol, pcol, bcol, acol):
  full_row = pl.BlockSpec((1, NPAD), lambda i: (0, 0))
  col = pl.BlockSpec((BI, 1), lambda i: (i, 0))
  return pl.pallas_call(
      _rank_body,
      grid=(NB,),
      in_specs=[full_row, full_row, full_row, full_row, col, col, col, col],
      out_specs=[
          pl.BlockSpec((BI, 1), lambda i: (i, 0)),
          pl.BlockSpec((1, G), lambda i: (0, 0)),
      ],
      out_shape=[
          jax.ShapeDtypeStruct((NPAD, 1), jnp.float32),
          jax.ShapeDtypeStruct((1, G), jnp.float32),
      ],
  )(srow, prow, brow, arow, scol, pcol, bcol, acol)


def _select_body(hmid, scol, rcol, bcol, acol, counts,
                 hnew_ref, kept16_ref, gmax_ref, gaps_ref, cnt_ref):
  i = pl.program_id(0)
  kvec = jnp.ceil(0.5 * counts[...])
  b = bcol[...]
  gids = lax.broadcasted_iota(jnp.float32, (1, G), 1)
  onehot = b == gids
  k_i = jnp.sum(jnp.where(onehot, kvec, 0.0), axis=1, keepdims=True)
  keptn = (acol[...] > 0.5) & (rcol[...] < k_i)
  keptf = jnp.where(keptn, 1.0, 0.0)
  hnew = hmid[...] * scol[...] * keptf
  hnew_ref[...] = hnew
  kept16_ref[...] = jnp.broadcast_to(keptf, (BI, 16))
  onehot_f = jnp.where(onehot & keptn, 1.0, 0.0)
  gaps = lax.dot_general(onehot_f, hnew, (((0,), (0,)), ((), ())),
                         preferred_element_type=jnp.float32)
  cnt = jnp.sum(onehot_f, axis=0)[:, None]

  @pl.when(i == 0)
  def _():
    gaps_ref[...] = jnp.zeros_like(gaps_ref)
    cnt_ref[...] = jnp.zeros_like(cnt_ref)
    gmax_ref[...] = jnp.full_like(gmax_ref, -3e38)

  gaps_ref[...] += gaps
  cnt_ref[...] += cnt
  gmin = jnp.min(b).astype(jnp.int32)
  gmax = jnp.max(b).astype(jnp.int32)

  def gbody(g, carry):
    mask = (b == g.astype(jnp.float32)) & keptn
    gm = jnp.max(jnp.where(mask, hnew, -3e38), axis=0, keepdims=True)
    cur = gmax_ref[pl.ds(g, 1), :]
    gmax_ref[pl.ds(g, 1), :] = jnp.maximum(cur, gm)
    return carry

  lax.fori_loop(gmin, gmax + 1, gbody, 0)


def _select_step(hmid, scol, rcol, bcol, acol, counts):
  col = pl.BlockSpec((BI, 1), lambda i: (i, 0))
  return pl.pallas_call(
      _select_body,
      grid=(NB,),
      in_specs=[
          pl.BlockSpec((BI, F), lambda i: (i, 0)),
          col, col, col, col,
          pl.BlockSpec((1, G), lambda i: (0, 0)),
      ],
      out_specs=[
          pl.BlockSpec((BI, F), lambda i: (i, 0)),
          pl.BlockSpec((BI, 16), lambda i: (i, 0)),
          pl.BlockSpec((G, F), lambda i: (0, 0)),
          pl.BlockSpec((G, F), lambda i: (0, 0)),
          pl.BlockSpec((G, 1), lambda i: (0, 0)),
      ],
      out_shape=[
          jax.ShapeDtypeStruct((NPAD, F), jnp.float32),
          jax.ShapeDtypeStruct((NPAD, 16), jnp.float32),
          jax.ShapeDtypeStruct((G, F), jnp.float32),
          jax.ShapeDtypeStruct((G, F), jnp.float32),
          jax.ShapeDtypeStruct((G, 1), jnp.float32),
      ],
  )(hmid, scol, rcol, bcol, acol, counts)


def _head_body(gm1, gs1, c1, gm2, gs2, c2, gm3, gs3, c3,
               l1w, l1b, l2w, l2b, l3w, l3b, out_ref):
  def readout(gm, gs, ct):
    c = ct[...]
    m = jnp.where(c > 0.5, gm[...], 0.0)
    a = gs[...] / jnp.maximum(c, 1.0)
    return jnp.concatenate([m, a], axis=1)

  z = readout(gm1, gs1, c1) + readout(gm2, gs2, c2) + readout(gm3, gs3, c3)
  z = jnp.maximum(
      lax.dot_general(z, l1w[...], (((1,), (1,)), ((), ())),
                      preferred_element_type=jnp.float32) + l1b[...], 0.0)
  z = jnp.maximum(
      lax.dot_general(z, l2w[...], (((1,), (1,)), ((), ())),
                      preferred_element_type=jnp.float32) + l2b[...], 0.0)
  z = lax.dot_general(z, l3w[...], (((1,), (1,)), ((), ())),
                      preferred_element_type=jnp.float32) + l3b[...]
  m = jnp.max(z, axis=1, keepdims=True)
  lse = m + jnp.log(jnp.sum(jnp.exp(z - m), axis=1, keepdims=True))
  out_ref[...] = z - lse


def _head_step(r1, r2, r3, l1w, l1b, l2w, l2b, l3w, l3b):
  return pl.pallas_call(
      _head_body,
      out_shape=jax.ShapeDtypeStruct((G, 10), jnp.float32),
  )(r1[0], r1[1], r1[2], r2[0], r2[1], r2[2], r3[0], r3[1], r3[2],
    l1w, l1b, l2w, l2b, l3w, l3b)


def kernel(x, edge_index, batch, W1l, b1l, W1r, W2l, b2l, W2r, W3l, b3l, W3r,
           p1w, p2w, p3w, lin1W, lin1b, lin2W, lin2b, lin3W, lin3b):
  f32 = jnp.float32
  src = edge_index[0]
  dst = edge_index[1]
  extra = NPAD - N
  pad_nodes = N + (jnp.arange(EPAD - E, dtype=jnp.int32) % extra)
  srcp = jnp.concatenate([src, pad_nodes])
  dstp = jnp.concatenate([dst, pad_nodes])
  xp = jnp.pad(x, ((0, extra), (0, 0)))
  batchp = jnp.pad(batch, (0, extra), constant_values=G - 1).astype(f32)
  act0 = (jnp.arange(NPAD) < N).astype(f32)
  kept16 = jnp.broadcast_to(act0[:, None], (NPAD, 16))
  zrow = jnp.zeros((NPAD, F), f32)
  zdeg = jnp.zeros((NPAD, 16), f32)
  bat_col = batchp[:, None]
  bat_row = batchp[None, :]
  pos_col = jnp.arange(NPAD, dtype=f32)[:, None]

  h = xp
  readouts = []
  for wl, bl, wr, pw in ((W1l, b1l, W1r, p1w),
                         (W2l, b2l, W2r, p2w),
                         (W3l, b3l, W3r, p3w)):
    s2, d2 = _sc_aggregate(h, srcp, dstp, kept16, zrow, zdeg)
    hmid, score = _layer_step(s2, d2, kept16, h, wl, bl[None, :], wr,
                              pw[None, :])
    act_col = kept16[:, :1]
    rank, counts = _rank_step(
        score.reshape(1, NPAD), pos_col.reshape(1, NPAD), bat_row,
        act_col.reshape(1, NPAD), score, pos_col, bat_col, act_col)
    h, kept16, gmax, gaps, cnt = _select_step(
        hmid, score, rank, bat_col, act_col, counts)
    pos_col = rank
    readouts.append((gmax, gaps, cnt))

  return _head_step(readouts[0], readouts[1], readouts[2],
                    lin1W, lin1b[None, :], lin2W, lin2b[None, :],
                    lin3W, lin3b[None, :])


# speedup vs baseline: 20.2127x; 20.2127x over previous
import functools

import jax
import jax.numpy as jnp
from jax import lax
from jax.experimental import pallas as pl
from jax.experimental.pallas import tpu as pltpu
from jax.experimental.pallas import tpu_sc as plsc

N = 10000
NPAD = 10240
E = 320000
F = 128
G = 64
NC, NS = 2, 16
NW = NC * NS
EPB = 128
CH = 80
EPAD = NW * EPB * CH
RPT = NPAD // NS
BLK = 512
BI = 512
WIN = 3 * BI
NB = NPAD // BI


_agg_out = (
    jax.ShapeDtypeStruct((NC, NPAD, F), jnp.float32),
    jax.ShapeDtypeStruct((NC, NPAD), jnp.float32),
)


def _sc_agg_body(h, srcp, dstp, kept1, zrow, zdeg, outs, outd,
                 idx_v, dst_v, rows_v, kf_v, kept_sh, acc_sh, deg_sh, gsem):
  c = lax.axis_index("c")
  s = lax.axis_index("s")
  wid = c * NS + s
  r0 = s * RPT
  pltpu.sync_copy(zrow.at[pl.ds(r0, RPT)], acc_sh.at[pl.ds(r0, RPT)])
  pltpu.sync_copy(zdeg.at[pl.ds(r0, RPT)], deg_sh.at[pl.ds(r0, RPT)])
  pltpu.sync_copy(kept1.at[pl.ds(r0, RPT)], kept_sh.at[pl.ds(r0, RPT)])
  plsc.subcore_barrier()
  ebase = wid * (CH * EPB)

  def chunk(i, carry):
    base = ebase + i * EPB
    pltpu.sync_copy(srcp.at[pl.ds(base, EPB)], idx_v)
    pltpu.sync_copy(dstp.at[pl.ds(base, EPB)], dst_v)
    pltpu.async_copy(h.at[idx_v], rows_v, gsem).wait()
    pltpu.sync_copy(kept_sh.at[idx_v], kf_v)
    pltpu.sync_copy(rows_v, acc_sh.at[dst_v], add=True)
    pltpu.sync_copy(kf_v, deg_sh.at[dst_v], add=True)
    return carry

  lax.fori_loop(0, CH, chunk, 0)
  plsc.subcore_barrier()
  pltpu.sync_copy(acc_sh.at[pl.ds(r0, RPT)], outs.at[c, pl.ds(r0, RPT)])
  pltpu.sync_copy(deg_sh.at[pl.ds(r0, RPT)], outd.at[c, pl.ds(r0, RPT)])


@functools.lru_cache(maxsize=None)
def _make_sc_aggregate():
  return pl.kernel(
      _sc_agg_body,
      out_type=_agg_out,
      mesh=plsc.VectorSubcoreMesh(core_axis_name="c", subcore_axis_name="s"),
      scratch_types=[
          pltpu.VMEM((EPB,), jnp.int32),
          pltpu.VMEM((EPB,), jnp.int32),
          pltpu.VMEM((EPB, F), jnp.float32),
          pltpu.VMEM((EPB,), jnp.float32),
          pltpu.VMEM_SHARED((NPAD,), jnp.float32),
          pltpu.VMEM_SHARED((NPAD, F), jnp.float32),
          pltpu.VMEM_SHARED((NPAD,), jnp.float32),
          pltpu.SemaphoreType.DMA,
      ],
  )


def _sc_aggregate(h, srcp, dstp, kept1, zrow, zdeg):
  return _make_sc_aggregate()(h, srcp, dstp, kept1, zrow, zdeg)


def _layer_body(s2, d2, kept, h, wl, bl, wr, pw, hmid_ref, score_ref):
  ssum = s2[0] + s2[1]
  deg = d2[0] + d2[1]
  keptc = kept[...]
  mean = ssum * keptc / jnp.maximum(deg, 1.0)
  z = (lax.dot_general(mean, wl[...], (((1,), (1,)), ((), ())),
                       preferred_element_type=jnp.float32)
       + bl[...]
       + lax.dot_general(h[...], wr[...], (((1,), (1,)), ((), ())),
                         preferred_element_type=jnp.float32))
  hmid = jnp.maximum(z, 0.0)
  hmid_ref[...] = hmid
  pwv = pw[...]
  nrm = jnp.sqrt(jnp.sum(pwv * pwv))
  score_ref[...] = jnp.tanh(
      jnp.sum(hmid * pwv, axis=1, keepdims=True) / nrm)


def _layer_step(s2, d2c, keptc, h, wl, bl, wr, pw):
  return pl.pallas_call(
      _layer_body,
      grid=(NPAD // BLK,),
      in_specs=[
          pl.BlockSpec((NC, BLK, F), lambda i: (0, i, 0)),
          pl.BlockSpec((NC, BLK, 1), lambda i: (0, i, 0)),
          pl.BlockSpec((BLK, 1), lambda i: (i, 0)),
          pl.BlockSpec((BLK, F), lambda i: (i, 0)),
          pl.BlockSpec((F, F), lambda i: (0, 0)),
          pl.BlockSpec((1, F), lambda i: (0, 0)),
          pl.BlockSpec((F, F), lambda i: (0, 0)),
          pl.BlockSpec((1, F), lambda i: (0, 0)),
      ],
      out_specs=[
          pl.BlockSpec((BLK, F), lambda i: (i, 0)),
          pl.BlockSpec((BLK, 1), lambda i: (i, 0)),
      ],
      out_shape=[
          jax.ShapeDtypeStruct((NPAD, F), jnp.float32),
          jax.ShapeDtypeStruct((NPAD, 1), jnp.float32),
      ],
  )(s2, d2c, keptc, h, wl, bl, wr, pw)


def _rank_body(srow, prow, brow, arow, scol, pcol, bcol, acol,
               rank_ref, counts_ref):
  i = pl.program_id(0)
  start = pl.multiple_of(jnp.clip(i * BI - BI, 0, NPAD - WIN), 128)
  sj = srow[:, pl.ds(start, WIN)]
  pj = prow[:, pl.ds(start, WIN)]
  bj = brow[:, pl.ds(start, WIN)]
  aj = arow[:, pl.ds(start, WIN)]
  si = scol[...]
  pi = pcol[...]
  bi = bcol[...]
  beats = (sj > si) | ((sj == si) & (pj < pi))
  valid = (bj == bi) & (aj > 0.5)
  rank_ref[...] = jnp.sum(
      jnp.where(beats & valid, 1.0, 0.0), axis=1, keepdims=True)
  gids = lax.broadcasted_iota(jnp.int32, (1, G), 1).astype(jnp.float32)
  cnt = jnp.sum(jnp.where((bi == gids) & (acol[...] > 0.5), 1.0, 0.0),
                axis=0, keepdims=True)

  @pl.when(i == 0)
  def _():
    counts_ref[...] = jnp.zeros_like(counts_ref)

  counts_ref[...] += cnt


def _rank_step(srow, prow, brow, arow, scol, pcol, bcol, acol):
  full_row = pl.BlockSpec((1, NPAD), lambda i: (0, 0))
  col = pl.BlockSpec((BI, 1), lambda i: (i, 0))
  return pl.pallas_call(
      _rank_body,
      grid=(NB,),
      in_specs=[full_row, full_row, full_row, full_row, col, col, col, col],
      out_specs=[
          pl.BlockSpec((BI, 1), lambda i: (i, 0)),
          pl.BlockSpec((1, G), lambda i: (0, 0)),
      ],
      out_shape=[
          jax.ShapeDtypeStruct((NPAD, 1), jnp.float32),
          jax.ShapeDtypeStruct((1, G), jnp.float32),
      ],
  )(srow, prow, brow, arow, scol, pcol, bcol, acol)


def _select_body(hmid, scol, rcol, bcol, acol, counts,
                 hnew_ref, keptc_ref, gmax_ref, gaps_ref, cnt_ref):
  i = pl.program_id(0)
  kvec = jnp.ceil(0.5 * counts[...])
  b = bcol[...]
  gids = lax.broadcasted_iota(jnp.int32, (1, G), 1).astype(jnp.float32)
  onehot = b == gids
  k_i = jnp.sum(jnp.where(onehot, kvec, 0.0), axis=1, keepdims=True)
  keptn = (acol[...] > 0.5) & (rcol[...] < k_i)
  keptf = jnp.where(keptn, 1.0, 0.0)
  hnew = hmid[...] * scol[...] * keptf
  hnew_ref[...] = hnew
  keptc_ref[...] = keptf
  onehot_f = jnp.where(onehot & keptn, 1.0, 0.0)
  gaps = lax.dot_general(onehot_f, hnew, (((0,), (0,)), ((), ())),
                         preferred_element_type=jnp.float32)
  cnt = jnp.sum(onehot_f, axis=0)[:, None]

  @pl.when(i == 0)
  def _():
    gaps_ref[...] = jnp.zeros_like(gaps_ref)
    cnt_ref[...] = jnp.zeros_like(cnt_ref)
    gmax_ref[...] = jnp.full_like(gmax_ref, -3e38)

  gaps_ref[...] += gaps
  cnt_ref[...] += cnt
  gmin = jnp.min(b).astype(jnp.int32)
  gmax = jnp.max(b).astype(jnp.int32)

  def gbody(g, carry):
    mask = (b == g.astype(jnp.float32)) & keptn
    gm = jnp.max(jnp.where(mask, hnew, -3e38), axis=0, keepdims=True)
    cur = gmax_ref[pl.ds(g, 1), :]
    gmax_ref[pl.ds(g, 1), :] = jnp.maximum(cur, gm)
    return carry

  lax.fori_loop(gmin, gmax + 1, gbody, 0)


def _select_step(hmid, scol, rcol, bcol, acol, counts):
  col = pl.BlockSpec((BI, 1), lambda i: (i, 0))
  return pl.pallas_call(
      _select_body,
      grid=(NB,),
      in_specs=[
          pl.BlockSpec((BI, F), lambda i: (i, 0)),
          col, col, col, col,
          pl.BlockSpec((1, G), lambda i: (0, 0)),
      ],
      out_specs=[
          pl.BlockSpec((BI, F), lambda i: (i, 0)),
          pl.BlockSpec((BI, 1), lambda i: (i, 0)),
          pl.BlockSpec((G, F), lambda i: (0, 0)),
          pl.BlockSpec((G, F), lambda i: (0, 0)),
          pl.BlockSpec((G, 1), lambda i: (0, 0)),
      ],
      out_shape=[
          jax.ShapeDtypeStruct((NPAD, F), jnp.float32),
          jax.ShapeDtypeStruct((NPAD, 1), jnp.float32),
          jax.ShapeDtypeStruct((G, F), jnp.float32),
          jax.ShapeDtypeStruct((G, F), jnp.float32),
          jax.ShapeDtypeStruct((G, 1), jnp.float32),
      ],
  )(hmid, scol, rcol, bcol, acol, counts)


def _head_body(gm1, gs1, c1, gm2, gs2, c2, gm3, gs3, c3,
               l1w, l1b, l2w, l2b, l3w, l3b, out_ref):
  def readout(gm, gs, ct):
    c = ct[...]
    m = jnp.where(c > 0.5, gm[...], 0.0)
    a = gs[...] / jnp.maximum(c, 1.0)
    return jnp.concatenate([m, a], axis=1)

  z = readout(gm1, gs1, c1) + readout(gm2, gs2, c2) + readout(gm3, gs3, c3)
  z = jnp.maximum(
      lax.dot_general(z, l1w[...], (((1,), (1,)), ((), ())),
                      preferred_element_type=jnp.float32) + l1b[...], 0.0)
  z = jnp.maximum(
      lax.dot_general(z, l2w[...], (((1,), (1,)), ((), ())),
                      preferred_element_type=jnp.float32) + l2b[...], 0.0)
  z = lax.dot_general(z, l3w[...], (((1,), (1,)), ((), ())),
                      preferred_element_type=jnp.float32) + l3b[...]
  m = jnp.max(z, axis=1, keepdims=True)
  lse = m + jnp.log(jnp.sum(jnp.exp(z - m), axis=1, keepdims=True))
  out_ref[...] = z - lse


def _head_step(r1, r2, r3, l1w, l1b, l2w, l2b, l3w, l3b):
  return pl.pallas_call(
      _head_body,
      out_shape=jax.ShapeDtypeStruct((G, 10), jnp.float32),
  )(r1[0], r1[1], r1[2], r2[0], r2[1], r2[2], r3[0], r3[1], r3[2],
    l1w, l1b, l2w, l2b, l3w, l3b)


def kernel(x, edge_index, batch, W1l, b1l, W1r, W2l, b2l, W2r, W3l, b3l, W3r,
           p1w, p2w, p3w, lin1W, lin1b, lin2W, lin2b, lin3W, lin3b):
  f32 = jnp.float32
  src = edge_index[0]
  dst = edge_index[1]
  extra = NPAD - N
  pad_nodes = N + (jnp.arange(EPAD - E, dtype=jnp.int32) % extra)
  srcp = jnp.concatenate([src, pad_nodes])
  dstp = jnp.concatenate([dst, pad_nodes])
  xp = jnp.pad(x, ((0, extra), (0, 0)))
  batchp = jnp.pad(batch, (0, extra), constant_values=G - 1).astype(f32)
  keptc = (jnp.arange(NPAD) < N).astype(f32)[:, None]
  zrow = jnp.zeros((NPAD, F), f32)
  zdeg = jnp.zeros((NPAD,), f32)
  bat_col = batchp[:, None]
  bat_row = batchp[None, :]
  pos_col = jnp.arange(NPAD, dtype=f32)[:, None]

  h = xp
  readouts = []
  for wl, bl, wr, pw in ((W1l, b1l, W1r, p1w),
                         (W2l, b2l, W2r, p2w),
                         (W3l, b3l, W3r, p3w)):
    s2, d2 = _sc_aggregate(h, srcp, dstp, keptc.reshape(NPAD), zrow, zdeg)
    hmid, score = _layer_step(s2, d2.reshape(NC, NPAD, 1), keptc, h, wl,
                              bl[None, :], wr, pw[None, :])
    rank, counts = _rank_step(
        score.reshape(1, NPAD), pos_col.reshape(1, NPAD), bat_row,
        keptc.reshape(1, NPAD), score, pos_col, bat_col, keptc)
    h, keptc, gmax, gaps, cnt = _select_step(
        hmid, score, rank, bat_col, keptc, counts)
    pos_col = rank
    readouts.append((gmax, gaps, cnt))

  return _head_step(readouts[0], readouts[1], readouts[2],
                    lin1W, lin1b[None, :], lin2W, lin2b[None, :],
                    lin3W, lin3b[None, :])


# speedup vs baseline: 31.0729x; 1.5373x over previous
import functools

import jax
import jax.numpy as jnp
from jax import lax
from jax.experimental import pallas as pl
from jax.experimental.pallas import tpu as pltpu
from jax.experimental.pallas import tpu_sc as plsc

N = 10000
NPAD = 10240
E = 320000
F = 128
G = 64
NC, NS = 2, 16
NW = NC * NS
EPB = 128
CH = 80
EPAD = NW * EPB * CH
RPT = NPAD // NS
BLK = 512
BI = 512
WIN = 3 * BI
NB = NPAD // BI


_agg_out = (
    jax.ShapeDtypeStruct((NC, NPAD, F), jnp.float32),
    jax.ShapeDtypeStruct((NC, NPAD), jnp.float32),
)


def _sc_agg_body(h, srcp, dstp, kept1, zrow, zdeg, outs, outd,
                 src_v, dst_v, rows_v, kf_v, kept_sh, acc_sh, deg_sh,
                 isem0, isem1, gsem0, gsem1, ksem0, ksem1):
  c = lax.axis_index("c")
  s = lax.axis_index("s")
  wid = c * NS + s
  r0 = s * RPT
  pltpu.sync_copy(zrow.at[pl.ds(r0, RPT)], acc_sh.at[pl.ds(r0, RPT)])
  pltpu.sync_copy(zdeg.at[pl.ds(r0, RPT)], deg_sh.at[pl.ds(r0, RPT)])
  pltpu.sync_copy(kept1.at[pl.ds(r0, RPT)], kept_sh.at[pl.ds(r0, RPT)])
  ebase = wid * (CH * EPB)

  isems = (isem0, isem1)
  gsems = (gsem0, gsem1)
  ksems = (ksem0, ksem1)

  def start_idx(ci, b):
    base = ebase + ci * EPB
    pltpu.async_copy(srcp.at[pl.ds(base, EPB)], src_v.at[b], isems[b])
    pltpu.async_copy(dstp.at[pl.ds(base, EPB)], dst_v.at[b], isems[b])

  def wait_idx(b):
    pltpu.make_async_copy(srcp.at[pl.ds(0, EPB)], src_v.at[b],
                          isems[b]).wait()
    pltpu.make_async_copy(dstp.at[pl.ds(0, EPB)], dst_v.at[b],
                          isems[b]).wait()

  def start_gather(b):
    pltpu.async_copy(h.at[src_v.at[b]], rows_v.at[b], gsems[b])
    pltpu.async_copy(kept_sh.at[src_v.at[b]], kf_v.at[b], ksems[b])

  def wait_gather_scatter(b):
    pltpu.make_async_copy(h.at[src_v.at[b]], rows_v.at[b], gsems[b]).wait()
    pltpu.make_async_copy(kept_sh.at[src_v.at[b]], kf_v.at[b],
                          ksems[b]).wait()
    pltpu.sync_copy(rows_v.at[b], acc_sh.at[dst_v.at[b]], add=True)
    pltpu.sync_copy(kf_v.at[b], deg_sh.at[dst_v.at[b]], add=True)

  plsc.subcore_barrier()
  start_idx(0, 0)
  start_idx(1, 1)
  wait_idx(0)
  start_gather(0)

  def step(ci, b, b1):
    @pl.when(ci + 1 < CH)
    def _():
      wait_idx(b1)
      start_gather(b1)

    wait_gather_scatter(b)

    @pl.when(ci + 2 < CH)
    def _():
      start_idx(ci + 2, b)

  def body(j, carry):
    c0 = 2 * j
    step(c0, 0, 1)
    step(c0 + 1, 1, 0)
    return carry

  lax.fori_loop(0, CH // 2, body, 0)
  plsc.subcore_barrier()
  pltpu.sync_copy(acc_sh.at[pl.ds(r0, RPT)], outs.at[c, pl.ds(r0, RPT)])
  pltpu.sync_copy(deg_sh.at[pl.ds(r0, RPT)], outd.at[c, pl.ds(r0, RPT)])


@functools.lru_cache(maxsize=None)
def _make_sc_aggregate():
  return pl.kernel(
      _sc_agg_body,
      out_type=_agg_out,
      mesh=plsc.VectorSubcoreMesh(core_axis_name="c", subcore_axis_name="s"),
      scratch_types=[
          pltpu.VMEM((2, EPB), jnp.int32),
          pltpu.VMEM((2, EPB), jnp.int32),
          pltpu.VMEM((2, EPB, F), jnp.float32),
          pltpu.VMEM((2, EPB), jnp.float32),
          pltpu.VMEM_SHARED((NPAD,), jnp.float32),
          pltpu.VMEM_SHARED((NPAD, F), jnp.float32),
          pltpu.VMEM_SHARED((NPAD,), jnp.float32),
          pltpu.SemaphoreType.DMA,
          pltpu.SemaphoreType.DMA,
          pltpu.SemaphoreType.DMA,
          pltpu.SemaphoreType.DMA,
          pltpu.SemaphoreType.DMA,
          pltpu.SemaphoreType.DMA,
      ],
  )


def _sc_aggregate(h, srcp, dstp, kept1, zrow, zdeg):
  return _make_sc_aggregate()(h, srcp, dstp, kept1, zrow, zdeg)


def _layer_body(s2, d2, kept, h, wl, bl, wr, pw, hmid_ref, score_ref):
  ssum = s2[0] + s2[1]
  deg = d2[0] + d2[1]
  keptc = kept[...]
  mean = ssum * keptc / jnp.maximum(deg, 1.0)
  z = (lax.dot_general(mean, wl[...], (((1,), (1,)), ((), ())),
                       preferred_element_type=jnp.float32)
       + bl[...]
       + lax.dot_general(h[...], wr[...], (((1,), (1,)), ((), ())),
                         preferred_element_type=jnp.float32))
  hmid = jnp.maximum(z, 0.0)
  hmid_ref[...] = hmid
  pwv = pw[...]
  nrm = jnp.sqrt(jnp.sum(pwv * pwv))
  score_ref[...] = jnp.tanh(
      jnp.sum(hmid * pwv, axis=1, keepdims=True) / nrm)


def _layer_step(s2, d2c, keptc, h, wl, bl, wr, pw):
  return pl.pallas_call(
      _layer_body,
      grid=(NPAD // BLK,),
      in_specs=[
          pl.BlockSpec((NC, BLK, F), lambda i: (0, i, 0)),
          pl.BlockSpec((NC, BLK, 1), lambda i: (0, i, 0)),
          pl.BlockSpec((BLK, 1), lambda i: (i, 0)),
          pl.BlockSpec((BLK, F), lambda i: (i, 0)),
          pl.BlockSpec((F, F), lambda i: (0, 0)),
          pl.BlockSpec((1, F), lambda i: (0, 0)),
          pl.BlockSpec((F, F), lambda i: (0, 0)),
          pl.BlockSpec((1, F), lambda i: (0, 0)),
      ],
      out_specs=[
          pl.BlockSpec((BLK, F), lambda i: (i, 0)),
          pl.BlockSpec((BLK, 1), lambda i: (i, 0)),
      ],
      out_shape=[
          jax.ShapeDtypeStruct((NPAD, F), jnp.float32),
          jax.ShapeDtypeStruct((NPAD, 1), jnp.float32),
      ],
  )(s2, d2c, keptc, h, wl, bl, wr, pw)


def _rank_body(srow, prow, brow, arow, scol, pcol, bcol, acol,
               rank_ref, counts_ref):
  i = pl.program_id(0)
  start = pl.multiple_of(jnp.clip(i * BI - BI, 0, NPAD - WIN), 128)
  sj = srow[:, pl.ds(start, WIN)]
  pj = prow[:, pl.ds(start, WIN)]
  bj = brow[:, pl.ds(start, WIN)]
  aj = arow[:, pl.ds(start, WIN)]
  si = scol[...]
  pi = pcol[...]
  bi = bcol[...]
  beats = (sj > si) | ((sj == si) & (pj < pi))
  valid = (bj == bi) & (aj > 0.5)
  rank_ref[...] = jnp.sum(
      jnp.where(beats & valid, 1.0, 0.0), axis=1, keepdims=True)
  gids = lax.broadcasted_iota(jnp.int32, (1, G), 1).astype(jnp.float32)
  cnt = jnp.sum(jnp.where((bi == gids) & (acol[...] > 0.5), 1.0, 0.0),
                axis=0, keepdims=True)

  @pl.when(i == 0)
  def _():
    counts_ref[...] = jnp.zeros_like(counts_ref)

  counts_ref[...] += cnt


def _rank_step(srow, prow, brow, arow, scol, pcol, bcol, acol):
  full_row = pl.BlockSpec((1, NPAD), lambda i: (0, 0))
  col = pl.BlockSpec((BI, 1), lambda i: (i, 0))
  return pl.pallas_call(
      _rank_body,
      grid=(NB,),
      in_specs=[full_row, full_row, full_row, full_row, col, col, col, col],
      out_specs=[
          pl.BlockSpec((BI, 1), lambda i: (i, 0)),
          pl.BlockSpec((1, G), lambda i: (0, 0)),
      ],
      out_shape=[
          jax.ShapeDtypeStruct((NPAD, 1), jnp.float32),
          jax.ShapeDtypeStruct((1, G), jnp.float32),
      ],
  )(srow, prow, brow, arow, scol, pcol, bcol, acol)


def _select_body(hmid, scol, rcol, bcol, acol, counts,
                 hnew_ref, keptc_ref, gmax_ref, gaps_ref, cnt_ref):
  i = pl.program_id(0)
  kvec = jnp.ceil(0.5 * counts[...])
  b = bcol[...]
  gids = lax.broadcasted_iota(jnp.int32, (1, G), 1).astype(jnp.float32)
  onehot = b == gids
  k_i = jnp.sum(jnp.where(onehot, kvec, 0.0), axis=1, keepdims=True)
  keptn = (acol[...] > 0.5) & (rcol[...] < k_i)
  keptf = jnp.where(keptn, 1.0, 0.0)
  hnew = hmid[...] * scol[...] * keptf
  hnew_ref[...] = hnew
  keptc_ref[...] = keptf
  onehot_f = jnp.where(onehot & keptn, 1.0, 0.0)
  gaps = lax.dot_general(onehot_f, hnew, (((0,), (0,)), ((), ())),
                         preferred_element_type=jnp.float32)
  cnt = jnp.sum(onehot_f, axis=0)[:, None]

  @pl.when(i == 0)
  def _():
    gaps_ref[...] = jnp.zeros_like(gaps_ref)
    cnt_ref[...] = jnp.zeros_like(cnt_ref)
    gmax_ref[...] = jnp.full_like(gmax_ref, -3e38)

  gaps_ref[...] += gaps
  cnt_ref[...] += cnt
  gmin = jnp.min(b).astype(jnp.int32)
  gmax = jnp.max(b).astype(jnp.int32)

  def gbody(g, carry):
    mask = (b == g.astype(jnp.float32)) & keptn
    gm = jnp.max(jnp.where(mask, hnew, -3e38), axis=0, keepdims=True)
    cur = gmax_ref[pl.ds(g, 1), :]
    gmax_ref[pl.ds(g, 1), :] = jnp.maximum(cur, gm)
    return carry

  lax.fori_loop(gmin, gmax + 1, gbody, 0)


def _select_step(hmid, scol, rcol, bcol, acol, counts):
  col = pl.BlockSpec((BI, 1), lambda i: (i, 0))
  return pl.pallas_call(
      _select_body,
      grid=(NB,),
      in_specs=[
          pl.BlockSpec((BI, F), lambda i: (i, 0)),
          col, col, col, col,
          pl.BlockSpec((1, G), lambda i: (0, 0)),
      ],
      out_specs=[
          pl.BlockSpec((BI, F), lambda i: (i, 0)),
          pl.BlockSpec((BI, 1), lambda i: (i, 0)),
          pl.BlockSpec((G, F), lambda i: (0, 0)),
          pl.BlockSpec((G, F), lambda i: (0, 0)),
          pl.BlockSpec((G, 1), lambda i: (0, 0)),
      ],
      out_shape=[
          jax.ShapeDtypeStruct((NPAD, F), jnp.float32),
          jax.ShapeDtypeStruct((NPAD, 1), jnp.float32),
          jax.ShapeDtypeStruct((G, F), jnp.float32),
          jax.ShapeDtypeStruct((G, F), jnp.float32),
          jax.ShapeDtypeStruct((G, 1), jnp.float32),
      ],
  )(hmid, scol, rcol, bcol, acol, counts)


def _head_body(gm1, gs1, c1, gm2, gs2, c2, gm3, gs3, c3,
               l1w, l1b, l2w, l2b, l3w, l3b, out_ref):
  def readout(gm, gs, ct):
    c = ct[...]
    m = jnp.where(c > 0.5, gm[...], 0.0)
    a = gs[...] / jnp.maximum(c, 1.0)
    return jnp.concatenate([m, a], axis=1)

  z = readout(gm1, gs1, c1) + readout(gm2, gs2, c2) + readout(gm3, gs3, c3)
  z = jnp.maximum(
      lax.dot_general(z, l1w[...], (((1,), (1,)), ((), ())),
                      preferred_element_type=jnp.float32) + l1b[...], 0.0)
  z = jnp.maximum(
      lax.dot_general(z, l2w[...], (((1,), (1,)), ((), ())),
                      preferred_element_type=jnp.float32) + l2b[...], 0.0)
  z = lax.dot_general(z, l3w[...], (((1,), (1,)), ((), ())),
                      preferred_element_type=jnp.float32) + l3b[...]
  m = jnp.max(z, axis=1, keepdims=True)
  lse = m + jnp.log(jnp.sum(jnp.exp(z - m), axis=1, keepdims=True))
  out_ref[...] = z - lse


def _head_step(r1, r2, r3, l1w, l1b, l2w, l2b, l3w, l3b):
  return pl.pallas_call(
      _head_body,
      out_shape=jax.ShapeDtypeStruct((G, 10), jnp.float32),
  )(r1[0], r1[1], r1[2], r2[0], r2[1], r2[2], r3[0], r3[1], r3[2],
    l1w, l1b, l2w, l2b, l3w, l3b)


def kernel(x, edge_index, batch, W1l, b1l, W1r, W2l, b2l, W2r, W3l, b3l, W3r,
           p1w, p2w, p3w, lin1W, lin1b, lin2W, lin2b, lin3W, lin3b):
  f32 = jnp.float32
  src = edge_index[0]
  dst = edge_index[1]
  extra = NPAD - N
  pad_nodes = N + (jnp.arange(EPAD - E, dtype=jnp.int32) % extra)
  srcp = jnp.concatenate([src, pad_nodes])
  dstp = jnp.concatenate([dst, pad_nodes])
  xp = jnp.pad(x, ((0, extra), (0, 0)))
  batchp = jnp.pad(batch, (0, extra), constant_values=G - 1).astype(f32)
  keptc = (jnp.arange(NPAD) < N).astype(f32)[:, None]
  zrow = jnp.zeros((NPAD, F), f32)
  zdeg = jnp.zeros((NPAD,), f32)
  bat_col = batchp[:, None]
  bat_row = batchp[None, :]
  pos_col = jnp.arange(NPAD, dtype=f32)[:, None]

  h = xp
  readouts = []
  for wl, bl, wr, pw in ((W1l, b1l, W1r, p1w),
                         (W2l, b2l, W2r, p2w),
                         (W3l, b3l, W3r, p3w)):
    s2, d2 = _sc_aggregate(h, srcp, dstp, keptc.reshape(NPAD), zrow, zdeg)
    hmid, score = _layer_step(s2, d2.reshape(NC, NPAD, 1), keptc, h, wl,
                              bl[None, :], wr, pw[None, :])
    rank, counts = _rank_step(
        score.reshape(1, NPAD), pos_col.reshape(1, NPAD), bat_row,
        keptc.reshape(1, NPAD), score, pos_col, bat_col, keptc)
    h, keptc, gmax, gaps, cnt = _select_step(
        hmid, score, rank, bat_col, keptc, counts)
    pos_col = rank
    readouts.append((gmax, gaps, cnt))

  return _head_step(readouts[0], readouts[1], readouts[2],
                    lin1W, lin1b[None, :], lin2W, lin2b[None, :],
                    lin3W, lin3b[None, :])
